# baseline (speedup 1.0000x reference)
import functools

import jax
import jax.numpy as jnp
from jax import lax
from jax.experimental import pallas as pl
from jax.experimental.pallas import tpu as pltpu
from jax.experimental.pallas import tpu_sc as plsc

_NODES = 10000
_EDGES = 320000
_D = 128

_NC = 2
_NS = 16
_NW = _NC * _NS
_EPW = _EDGES // _NW
_B = 40
_NCH = _EPW // _B
_NG = 5
_CPG = _NCH // _NG
_EPG = _CPG * _B
_PAIRS = _CPG // 2
_NPAD = 10240
_RPS = _NPAD // _NS
_ZR = 32


def _sc_segment_sum(src3, dst4, dst3, feature):
    mesh = plsc.VectorSubcoreMesh(core_axis_name="c", subcore_axis_name="s")

    @functools.partial(
        pl.kernel,
        out_type=[
            jax.ShapeDtypeStruct((_NC, _NPAD, _D), jnp.float32),
            jax.ShapeDtypeStruct((_NC, _NS, 1, _NPAD), jnp.int32),
        ],
        mesh=mesh,
        compiler_params=pltpu.CompilerParams(needs_layout_passes=False),
        scratch_types=[
            pltpu.VMEM((_EPG,), jnp.int32),
            pltpu.VMEM((_EPG,), jnp.int32),
            pltpu.VMEM((_CPG, _B), jnp.int32),
            pltpu.VMEM((_CPG, _B), jnp.int32),
            pltpu.VMEM((_EPG,), jnp.int32),
            pltpu.VMEM((_EPG,), jnp.int32),
            pltpu.VMEM((2, _B, _D), jnp.float32),
            pltpu.VMEM((_ZR, _D), jnp.float32),
            pltpu.VMEM((_NPAD,), jnp.int32),
            pltpu.VMEM_SHARED((_NPAD, _D), jnp.float32),
            pltpu.SemaphoreType.DMA,
            pltpu.SemaphoreType.DMA,
            pltpu.SemaphoreType.DMA,
            pltpu.SemaphoreType.DMA,
            pltpu.SemaphoreType.DMA,
        ],
    )
    def sc(src_hbm, dst4_hbm, dst3_hbm, feat_hbm, acc_out, deg_out,
           sf0, sf1, d20, d21, df0, df1, rows_v, zf_v, deg_v, acc_sh,
           g0, g1, s0, s1, isem):
        c = lax.axis_index("c")
        s = lax.axis_index("s")
        wid = s * _NC + c
        gsem = (g0, g1)
        ssem = (s0, s1)
        sfb = (sf0, sf1)
        d2b = (d20, d21)
        dfb = (df0, df1)

        zero16f = jnp.zeros((16,), jnp.float32)
        zero16i = jnp.zeros((16,), jnp.int32)

        def init_zf(i, carry):
            for k in range(_D // 16):
                zf_v[i, pl.ds(16 * k, 16)] = zero16f
            return carry

        lax.fori_loop(0, _ZR, init_zf, 0)

        def init_deg(i, carry):
            deg_v[pl.ds(16 * i, 16)] = zero16i
            return carry

        lax.fori_loop(0, _NPAD // 16, init_deg, 0)

        def stage_start(g, p):
            pltpu.async_copy(src_hbm.at[wid, g, 0], sfb[p], isem)
            pltpu.async_copy(dst4_hbm.at[wid, g], d2b[p], isem)
            pltpu.async_copy(dst3_hbm.at[wid, g, 0], dfb[p], isem)

        def stage_wait(g, p):
            pltpu.make_async_copy(src_hbm.at[wid, g, 0], sfb[p], isem).wait()
            pltpu.make_async_copy(dst4_hbm.at[wid, g], d2b[p], isem).wait()
            pltpu.make_async_copy(dst3_hbm.at[wid, g, 0], dfb[p], isem).wait()

        def gather_start(p, j, slot):
            pltpu.async_copy(feat_hbm.at[sfb[p].at[pl.ds(j * _B, _B)]],
                             rows_v.at[slot], gsem[slot])

        def gather_wait(p, j, slot):
            pltpu.make_async_copy(feat_hbm.at[sfb[p].at[pl.ds(j * _B, _B)]],
                                  rows_v.at[slot], gsem[slot]).wait()

        def scatter_start(p, j, slot):
            pltpu.async_copy(rows_v.at[slot], acc_sh.at[d2b[p].at[j]],
                             ssem[slot], add=True)

        def scatter_wait(p, j, slot):
            pltpu.make_async_copy(rows_v.at[slot], acc_sh.at[d2b[p].at[j]],
                                  ssem[slot]).wait()

        stage_start(0, 0)
        stage_wait(0, 0)
        gather_start(0, 0, 0)
        gather_start(0, 1, 1)

        for k in range(_RPS // _ZR):
            pltpu.sync_copy(zf_v, acc_sh.at[pl.ds(s * _RPS + k * _ZR, _ZR)])
        plsc.subcore_barrier()

        for g in range(_NG):
            p = g & 1

            if g + 1 < _NG:
                stage_start(g + 1, 1 - p)

            def pair(i, carry):
                j0 = 2 * i
                j1 = 2 * i + 1
                gather_wait(p, j0, 0)
                scatter_start(p, j0, 0)
                gather_wait(p, j1, 1)
                scatter_start(p, j1, 1)

                for k in range(5):
                    ids = dfb[p][pl.ds(i * 80 + 16 * k, 16)]
                    cnt, last = plsc.scan_count(ids)
                    plsc.addupdate_scatter(deg_v, [ids], cnt, mask=last)

                scatter_wait(p, j0, 0)

                @pl.when(j0 + 2 < _CPG)
                def _():
                    gather_start(p, j0 + 2, 0)

                scatter_wait(p, j1, 1)

                @pl.when(j1 + 2 < _CPG)
                def _():
                    gather_start(p, j1 + 2, 1)

                return carry

            lax.fori_loop(0, _PAIRS, pair, 0)

            if g + 1 < _NG:
                stage_wait(g + 1, 1 - p)
                gather_start(1 - p, 0, 0)
                gather_start(1 - p, 1, 1)

        plsc.subcore_barrier()

        pltpu.sync_copy(acc_sh.at[pl.ds(s * _RPS, _RPS)],
                        acc_out.at[c, pl.ds(s * _RPS, _RPS)])
        pltpu.sync_copy(deg_v, deg_out.at[c, s, 0])

    return sc(src3, dst4, dst3, feature)


def _tc_finish(acc2, deg2, W, b2):
    def body(acc_ref, deg_ref, w_ref, b_ref, out_ref):
        a = acc_ref[0, :_NODES] + acc_ref[1, :_NODES]
        d = jnp.sum(deg_ref[...].astype(jnp.float32), axis=(0, 1, 2))
        d = jnp.reshape(jnp.maximum(d[:_NODES], 1.0), (_NODES, 1))
        h = a / d
        y = lax.dot_general(h, w_ref[...], (((1,), (1,)), ((), ())),
                            preferred_element_type=jnp.float32)
        out_ref[...] = jnp.maximum(y + b_ref[...], 0.0)

    return pl.pallas_call(
        body,
        out_shape=jax.ShapeDtypeStruct((_NODES, _D), jnp.float32),
    )(acc2, deg2, W, b2)


def kernel(feature, edge_index, W, b):
    src3 = edge_index[0].astype(jnp.int32).reshape(_NW, _NG, 1, _EPG)
    dst4 = edge_index[1].astype(jnp.int32).reshape(_NW, _NG, _CPG, _B)
    dst3 = edge_index[1].astype(jnp.int32).reshape(_NW, _NG, 1, _EPG)
    acc2, deg2 = _sc_segment_sum(src3, dst4, dst3, feature)
    return _tc_finish(acc2, deg2, W, b.reshape(1, _D))


# speedup vs baseline: 1.0265x; 1.0265x over previous
import functools

import jax
import jax.numpy as jnp
from jax import lax
from jax.experimental import pallas as pl
from jax.experimental.pallas import tpu as pltpu
from jax.experimental.pallas import tpu_sc as plsc

_NODES = 10000
_EDGES = 320000
_D = 128

_NC = 2
_NS = 16
_NW = _NC * _NS
_EPW = _EDGES // _NW
_B = 40
_NCH = _EPW // _B
_NG = 5
_CPG = _NCH // _NG
_EPG = _CPG * _B
_PAIRS = _CPG // 2
_NPAD = 10240
_RPS = _NPAD // _NS
_ZR = 32


def _sc_segment_sum(src3, dst3, feature):
    mesh = plsc.VectorSubcoreMesh(core_axis_name="c", subcore_axis_name="s")

    @functools.partial(
        pl.kernel,
        out_type=[
            jax.ShapeDtypeStruct((_NC, _NPAD, _D), jnp.float32),
            jax.ShapeDtypeStruct((_NC, _NPAD), jnp.int32),
        ],
        mesh=mesh,
        compiler_params=pltpu.CompilerParams(needs_layout_passes=False),
        scratch_types=[
            pltpu.VMEM((_EPG,), jnp.int32),
            pltpu.VMEM((_EPG,), jnp.int32),
            pltpu.VMEM((_EPG,), jnp.int32),
            pltpu.VMEM((_EPG,), jnp.int32),
            pltpu.VMEM((2, _B, _D), jnp.float32),
            pltpu.VMEM((_ZR, _D), jnp.float32),
            pltpu.VMEM((_NPAD,), jnp.int32),
            pltpu.VMEM((_RPS,), jnp.int32),
            pltpu.VMEM((_RPS,), jnp.int32),
            pltpu.VMEM_SHARED((_NPAD, _D), jnp.float32),
            pltpu.VMEM_SHARED((_NS, _NPAD), jnp.int32),
            pltpu.SemaphoreType.DMA,
            pltpu.SemaphoreType.DMA,
            pltpu.SemaphoreType.DMA,
            pltpu.SemaphoreType.DMA,
            pltpu.SemaphoreType.DMA,
        ],
    )
    def sc(src_hbm, dst3_hbm, feat_hbm, acc_out, deg_out,
           sf0, sf1, df0, df1, rows_v, zf_v, deg_v, dm_a, dm_b, acc_sh,
           deg_st, g0, g1, s0, s1, isem):
        c = lax.axis_index("c")
        s = lax.axis_index("s")
        wid = s * _NC + c
        gsem = (g0, g1)
        ssem = (s0, s1)
        sfb = (sf0, sf1)
        dfb = (df0, df1)

        zero16f = jnp.zeros((16,), jnp.float32)
        zero16i = jnp.zeros((16,), jnp.int32)

        def init_zf(i, carry):
            for k in range(_D // 16):
                zf_v[i, pl.ds(16 * k, 16)] = zero16f
            return carry

        lax.fori_loop(0, _ZR, init_zf, 0)

        def init_deg(i, carry):
            deg_v[pl.ds(16 * i, 16)] = zero16i
            return carry

        lax.fori_loop(0, _NPAD // 16, init_deg, 0)

        def stage_start(g, p):
            pltpu.async_copy(src_hbm.at[wid, g, 0], sfb[p], isem)
            pltpu.async_copy(dst3_hbm.at[wid, g, 0], dfb[p], isem)

        def stage_wait(g, p):
            pltpu.make_async_copy(src_hbm.at[wid, g, 0], sfb[p], isem).wait()
            pltpu.make_async_copy(dst3_hbm.at[wid, g, 0], dfb[p], isem).wait()

        def gather_start(p, j, slot):
            pltpu.async_copy(feat_hbm.at[sfb[p].at[pl.ds(j * _B, _B)]],
                             rows_v.at[slot], gsem[slot])

        def gather_wait(p, j, slot):
            pltpu.make_async_copy(feat_hbm.at[sfb[p].at[pl.ds(j * _B, _B)]],
                                  rows_v.at[slot], gsem[slot]).wait()

        def scatter_start(p, j, slot):
            pltpu.async_copy(rows_v.at[slot],
                             acc_sh.at[dfb[p].at[pl.ds(j * _B, _B)]],
                             ssem[slot], add=True)

        def scatter_wait(p, j, slot):
            pltpu.make_async_copy(rows_v.at[slot],
                                  acc_sh.at[dfb[p].at[pl.ds(j * _B, _B)]],
                                  ssem[slot]).wait()

        stage_start(0, 0)
        stage_wait(0, 0)
        gather_start(0, 0, 0)
        gather_start(0, 1, 1)

        for k in range(_RPS // _ZR):
            pltpu.sync_copy(zf_v, acc_sh.at[pl.ds(s * _RPS + k * _ZR, _ZR)])
        plsc.subcore_barrier()

        for g in range(_NG):
            p = g & 1

            if g + 1 < _NG:
                stage_start(g + 1, 1 - p)

            def pair(i, carry):
                j0 = 2 * i
                j1 = 2 * i + 1
                gather_wait(p, j0, 0)
                scatter_start(p, j0, 0)
                gather_wait(p, j1, 1)
                scatter_start(p, j1, 1)

                for k in range(5):
                    ids = dfb[p][pl.ds(i * 80 + 16 * k, 16)]
                    cnt, last = plsc.scan_count(ids)
                    plsc.addupdate_scatter(deg_v, [ids], cnt, mask=last)

                scatter_wait(p, j0, 0)

                @pl.when(j0 + 2 < _CPG)
                def _():
                    gather_start(p, j0 + 2, 0)

                scatter_wait(p, j1, 1)

                @pl.when(j1 + 2 < _CPG)
                def _():
                    gather_start(p, j1 + 2, 1)

                return carry

            lax.fori_loop(0, _PAIRS, pair, 0)

            if g + 1 < _NG:
                stage_wait(g + 1, 1 - p)
                gather_start(1 - p, 0, 0)
                gather_start(1 - p, 1, 1)

        pltpu.sync_copy(deg_v, deg_st.at[s])
        plsc.subcore_barrier()

        pltpu.async_copy(acc_sh.at[pl.ds(s * _RPS, _RPS)],
                         acc_out.at[c, pl.ds(s * _RPS, _RPS)], g0)
        pltpu.sync_copy(deg_st.at[0, pl.ds(s * _RPS, _RPS)], dm_a)
        for t in range(1, _NS):
            pltpu.sync_copy(deg_st.at[t, pl.ds(s * _RPS, _RPS)], dm_b)
            def acc_deg(i, carry):
                sl = pl.ds(16 * i, 16)
                dm_a[sl] = dm_a[sl] + dm_b[sl]
                return carry
            lax.fori_loop(0, _RPS // 16, acc_deg, 0)
        pltpu.sync_copy(dm_a, deg_out.at[c, pl.ds(s * _RPS, _RPS)])
        pltpu.make_async_copy(acc_sh.at[pl.ds(s * _RPS, _RPS)],
                              acc_out.at[c, pl.ds(s * _RPS, _RPS)], g0).wait()

    return sc(src3, dst3, feature)


def _tc_finish(acc2, deg2, W, b2):
    def body(acc_ref, deg_ref, w_ref, b_ref, out_ref):
        a = acc_ref[0, :_NODES] + acc_ref[1, :_NODES]
        d = (deg_ref[0] + deg_ref[1]).astype(jnp.float32)
        d = jnp.reshape(jnp.maximum(d[:_NODES], 1.0), (_NODES, 1))
        h = a / d
        y = lax.dot_general(h, w_ref[...], (((1,), (1,)), ((), ())),
                            preferred_element_type=jnp.float32)
        out_ref[...] = jnp.maximum(y + b_ref[...], 0.0)

    return pl.pallas_call(
        body,
        out_shape=jax.ShapeDtypeStruct((_NODES, _D), jnp.float32),
    )(acc2, deg2, W, b2)


def kernel(feature, edge_index, W, b):
    src3 = edge_index[0].astype(jnp.int32).reshape(_NW, _NG, 1, _EPG)
    dst3 = edge_index[1].astype(jnp.int32).reshape(_NW, _NG, 1, _EPG)
    acc2, deg2 = _sc_segment_sum(src3, dst3, feature)
    return _tc_finish(acc2, deg2, W, b.reshape(1, _D))


# speedup vs baseline: 1.1051x; 1.0765x over previous
import functools

import jax
import jax.numpy as jnp
from jax import lax
from jax.experimental import pallas as pl
from jax.experimental.pallas import tpu as pltpu
from jax.experimental.pallas import tpu_sc as plsc

_NODES = 10000
_EDGES = 320000
_D = 128

_NC = 2
_NS = 16
_NW = _NC * _NS
_EPW = _EDGES // _NW
_B = 40
_NCH = _EPW // _B
_NG = 5
_CPG = _NCH // _NG
_EPG = _CPG * _B
_PAIRS = _CPG // 2
_NPAD = 10240
_RPS = _NPAD // _NS
_ZR = 32


def _sc_segment_sum(src3, dst3, feature):
    mesh = plsc.VectorSubcoreMesh(core_axis_name="c", subcore_axis_name="s")

    @functools.partial(
        pl.kernel,
        out_type=[
            jax.ShapeDtypeStruct((_NC, _NPAD, _D), jnp.float32),
            jax.ShapeDtypeStruct((_NC, _NPAD), jnp.float32),
        ],
        mesh=mesh,
        scratch_types=[
            pltpu.VMEM((_EPG,), jnp.int32),
            pltpu.VMEM((_EPG,), jnp.int32),
            pltpu.VMEM((_EPG,), jnp.int32),
            pltpu.VMEM((_EPG,), jnp.int32),
            pltpu.VMEM((2, _B, _D), jnp.float32),
            pltpu.VMEM((_B,), jnp.float32),
            pltpu.VMEM((_ZR, _D), jnp.float32),
            pltpu.VMEM((_RPS,), jnp.float32),
            pltpu.VMEM_SHARED((_NPAD, _D), jnp.float32),
            pltpu.VMEM_SHARED((_NPAD,), jnp.float32),
            pltpu.SemaphoreType.DMA,
            pltpu.SemaphoreType.DMA,
            pltpu.SemaphoreType.DMA,
            pltpu.SemaphoreType.DMA,
            pltpu.SemaphoreType.DMA,
            pltpu.SemaphoreType.DMA,
            pltpu.SemaphoreType.DMA,
        ],
    )
    def sc(src_hbm, dst3_hbm, feat_hbm, acc_out, deg_out,
           sf0, sf1, df0, df1, rows_v, ones_v, zf_v, zd_v, acc_sh, deg_sh,
           g0, g1, s0, s1, d0, d1, isem):
        c = lax.axis_index("c")
        s = lax.axis_index("s")
        wid = s * _NC + c
        gsem = (g0, g1)
        ssem = (s0, s1)
        dsem = (d0, d1)
        sfb = (sf0, sf1)
        dfb = (df0, df1)

        zero16f = jnp.zeros((16,), jnp.float32)
        one16f = jnp.ones((16,), jnp.float32)

        for k in range(_B // 16):
            ones_v[pl.ds(16 * k, 16)] = one16f
        ones_v[pl.ds(_B - 16, 16)] = one16f

        def init_zf(i, carry):
            for k in range(_D // 16):
                zf_v[i, pl.ds(16 * k, 16)] = zero16f
            return carry

        lax.fori_loop(0, _ZR, init_zf, 0)

        def init_zd(i, carry):
            zd_v[pl.ds(16 * i, 16)] = zero16f
            return carry

        lax.fori_loop(0, _RPS // 16, init_zd, 0)

        def stage_start(g, p):
            pltpu.async_copy(src_hbm.at[wid, g, 0], sfb[p], isem)
            pltpu.async_copy(dst3_hbm.at[wid, g, 0], dfb[p], isem)

        def stage_wait(g, p):
            pltpu.make_async_copy(src_hbm.at[wid, g, 0], sfb[p], isem).wait()
            pltpu.make_async_copy(dst3_hbm.at[wid, g, 0], dfb[p], isem).wait()

        def gather_start(p, j, slot):
            pltpu.async_copy(feat_hbm.at[sfb[p].at[pl.ds(j * _B, _B)]],
                             rows_v.at[slot], gsem[slot])

        def gather_wait(p, j, slot):
            pltpu.make_async_copy(feat_hbm.at[sfb[p].at[pl.ds(j * _B, _B)]],
                                  rows_v.at[slot], gsem[slot]).wait()

        def scatter_start(p, j, slot):
            pltpu.async_copy(rows_v.at[slot],
                             acc_sh.at[dfb[p].at[pl.ds(j * _B, _B)]],
                             ssem[slot], add=True)

        def scatter_wait(p, j, slot):
            pltpu.make_async_copy(rows_v.at[slot],
                                  acc_sh.at[dfb[p].at[pl.ds(j * _B, _B)]],
                                  ssem[slot]).wait()

        def deg_start(p, j, slot):
            pltpu.async_copy(ones_v, deg_sh.at[dfb[p].at[pl.ds(j * _B, _B)]],
                             dsem[slot], add=True)

        def deg_wait(p, j, slot):
            pltpu.make_async_copy(ones_v,
                                  deg_sh.at[dfb[p].at[pl.ds(j * _B, _B)]],
                                  dsem[slot]).wait()

        stage_start(0, 0)
        stage_wait(0, 0)
        gather_start(0, 0, 0)
        gather_start(0, 1, 1)

        for k in range(_RPS // _ZR):
            pltpu.sync_copy(zf_v, acc_sh.at[pl.ds(s * _RPS + k * _ZR, _ZR)])
        pltpu.sync_copy(zd_v, deg_sh.at[pl.ds(s * _RPS, _RPS)])
        plsc.subcore_barrier()

        for g in range(_NG):
            p = g & 1

            if g + 1 < _NG:
                stage_start(g + 1, 1 - p)

            def pair(i, carry):
                j0 = 2 * i
                j1 = 2 * i + 1
                gather_wait(p, j0, 0)
                scatter_start(p, j0, 0)
                deg_start(p, j0, 0)
                gather_wait(p, j1, 1)
                scatter_start(p, j1, 1)
                deg_start(p, j1, 1)

                scatter_wait(p, j0, 0)
                deg_wait(p, j0, 0)

                @pl.when(j0 + 2 < _CPG)
                def _():
                    gather_start(p, j0 + 2, 0)

                scatter_wait(p, j1, 1)
                deg_wait(p, j1, 1)

                @pl.when(j1 + 2 < _CPG)
                def _():
                    gather_start(p, j1 + 2, 1)

                return carry

            lax.fori_loop(0, _PAIRS, pair, 0)

            if g + 1 < _NG:
                stage_wait(g + 1, 1 - p)
                gather_start(1 - p, 0, 0)
                gather_start(1 - p, 1, 1)

        plsc.subcore_barrier()

        pltpu.sync_copy(acc_sh.at[pl.ds(s * _RPS, _RPS)],
                        acc_out.at[c, pl.ds(s * _RPS, _RPS)])
        pltpu.sync_copy(deg_sh.at[pl.ds(s * _RPS, _RPS)],
                        deg_out.at[c, pl.ds(s * _RPS, _RPS)])

    return sc(src3, dst3, feature)


def _tc_finish(acc2, deg2, W, b2):
    def body(acc_ref, deg_ref, w_ref, b_ref, out_ref):
        a = acc_ref[0, :_NODES] + acc_ref[1, :_NODES]
        d = deg_ref[0] + deg_ref[1]
        d = jnp.reshape(jnp.maximum(d[:_NODES], 1.0), (_NODES, 1))
        h = a / d
        y = lax.dot_general(h, w_ref[...], (((1,), (1,)), ((), ())),
                            preferred_element_type=jnp.float32)
        out_ref[...] = jnp.maximum(y + b_ref[...], 0.0)

    return pl.pallas_call(
        body,
        out_shape=jax.ShapeDtypeStruct((_NODES, _D), jnp.float32),
    )(acc2, deg2, W, b2)


def kernel(feature, edge_index, W, b):
    src3 = edge_index[0].astype(jnp.int32).reshape(_NW, _NG, 1, _EPG)
    dst3 = edge_index[1].astype(jnp.int32).reshape(_NW, _NG, 1, _EPG)
    acc2, deg2 = _sc_segment_sum(src3, dst3, feature)
    return _tc_finish(acc2, deg2, W, b.reshape(1, _D))
